# baseline (speedup 1.0000x reference)
import dataclasses
import functools

import jax
import jax.numpy as jnp
from jax import lax
from jax.experimental import pallas as pl
from jax.experimental.pallas import tpu as pltpu
from jax.experimental.pallas import tpu_sc as plsc

N = 10000
P = 320000
NE1, NE2 = 1024, 2048
NS = 16
K = 128
PT = 20480
PP = NS * PT
NCH = PT // K
NE1P, NE2P = 1280, 2304
NVP = 10240

_MESH = plsc.VectorSubcoreMesh(core_axis_name="c", subcore_axis_name="s")
_f32 = jnp.float32


def _sds(shape):
    return jax.ShapeDtypeStruct(shape, _f32)


GRP = 8


def _seg_stream(tbl_hbm, gidx_hbm, sidx_hbm, out_hbm, z_hbm, acc_rows,
                exp_rows, acc_sh, gidx_a, sidx_a, bufs, gsems, ssems,
                sid):
    ar = acc_rows // NS
    pltpu.sync_copy(z_hbm.at[pl.ds(0, ar)], acc_sh.at[pl.ds(sid * ar, ar)])
    plsc.subcore_barrier()

    @pl.loop(0, NCH // GRP)
    def _(g):
        base = sid * NCH + g * GRP
        pltpu.sync_copy(gidx_hbm.at[pl.ds(base, GRP)], gidx_a)
        pltpu.sync_copy(sidx_hbm.at[pl.ds(base, GRP)], sidx_a)
        nb_ = len(bufs)
        gobj = [None] * GRP
        sobj = [None] * GRP
        for b in range(nb_):
            gobj[b] = pltpu.async_copy(tbl_hbm.at[gidx_a.at[b]],
                                       bufs[b], gsems[b])
        for b in range(GRP):
            i = b % nb_
            gobj[b].wait()
            sobj[b] = pltpu.async_copy(bufs[i], acc_sh.at[sidx_a.at[b]],
                                       ssems[i], add=True)
            nxt = b + nb_
            if nxt < GRP:
                sobj[b].wait()
                gobj[nxt] = pltpu.async_copy(tbl_hbm.at[gidx_a.at[nxt]],
                                             bufs[i], gsems[i])
        for b in range(GRP - nb_, GRP):
            sobj[b].wait()

    plsc.subcore_barrier()
    er = exp_rows // NS
    pltpu.sync_copy(acc_sh.at[pl.ds(sid * er, er)],
                    out_hbm.at[pl.ds(sid * er, er)])


def _make_seg_kernel(C, accA_rows, accB_rows, expA_rows, expB_rows, NB=4):
    acc_max = max(accA_rows, accB_rows)

    @functools.partial(
        pl.kernel,
        out_type=[_sds((expA_rows, C)), _sds((expB_rows, C))],
        mesh=_MESH,
        scratch_types=[
            pltpu.VMEM_SHARED((acc_max, C), _f32),
            pltpu.VMEM((GRP, K), jnp.int32),
            pltpu.VMEM((GRP, K), jnp.int32),
        ] + [pltpu.VMEM((K, C), _f32)] * NB
          + [pltpu.SemaphoreType.DMA] * (2 * NB),
    )
    def seg(tblA, gA, sA, tblB, gB, sB, zf, outA, outB,
            acc_sh, gidx_a, sidx_a, *rest):
        cid = lax.axis_index("c")
        sid = lax.axis_index("s")
        bufs = tuple(rest[:NB])
        gsems = tuple(rest[NB:2 * NB])
        ssems = tuple(rest[2 * NB:])

        @pl.when(cid == 0)
        def _():
            _seg_stream(tblA, gA, sA, outA, zf, accA_rows, expA_rows,
                        acc_sh, gidx_a, sidx_a, bufs, gsems, ssems, sid)

        @pl.when(cid == 1)
        def _():
            _seg_stream(tblB, gB, sB, outB, zf, accB_rows, expB_rows,
                        acc_sh, gidx_a, sidx_a, bufs, gsems, ssems, sid)

    return seg


_seg_edge = _make_seg_kernel(128, NE1P, NE2P, NE1P, NE2P)
_seg_vertex = _make_seg_kernel(128, NVP, NVP, NVP, NVP, NB=2)
_seg_attn = _seg_edge


def _deg_stream(vidx_hbm, eidx_hbm, dv_hbm, de_hbm, z_hbm, ones_v,
                de_rows, dv_sh, de_sh, vidx_v, eidx_v, semv, seme, sid):
    dvr = NVP // NS
    pltpu.sync_copy(z_hbm.at[pl.ds(0, dvr)], dv_sh.at[pl.ds(sid * dvr, dvr)])
    der = de_rows // NS
    pltpu.sync_copy(z_hbm.at[pl.ds(0, der)], de_sh.at[pl.ds(sid * der, der)])
    plsc.subcore_barrier()

    @pl.loop(0, NCH // GRP)
    def _(g):
        base = sid * NCH + g * GRP
        pltpu.sync_copy(vidx_hbm.at[pl.ds(base, GRP)], vidx_v)
        pltpu.sync_copy(eidx_hbm.at[pl.ds(base, GRP)], eidx_v)
        objs = []
        for b in range(GRP):
            objs.append(pltpu.async_copy(
                ones_v, dv_sh.at[vidx_v.at[b]], semv, add=True))
            objs.append(pltpu.async_copy(
                ones_v, de_sh.at[eidx_v.at[b]], seme, add=True))
        for o in objs:
            o.wait()

    plsc.subcore_barrier()
    pltpu.sync_copy(dv_sh.at[pl.ds(sid * dvr, dvr)],
                    dv_hbm.at[pl.ds(sid * dvr, dvr)])
    pltpu.sync_copy(de_sh.at[pl.ds(sid * der, der)],
                    de_hbm.at[pl.ds(sid * der, der)])


@functools.partial(
    pl.kernel,
    out_type=[_sds((NVP, 128)), _sds((NE1P, 128)), _sds((NVP, 128)), _sds((NE2P, 128))],
    mesh=_MESH,
    scratch_types=[
        pltpu.VMEM_SHARED((NVP, 128), _f32),
        pltpu.VMEM_SHARED((NE2P, 128), _f32),
        pltpu.VMEM((GRP, K), jnp.int32),
        pltpu.VMEM((GRP, K), jnp.int32),
        pltpu.VMEM((K, 128), _f32),
        pltpu.SemaphoreType.DMA,
        pltpu.SemaphoreType.DMA,
    ],
)
def _sc_degrees(v1, e1, v2, e2, ones_hbm, zf, dv1, de1, dv2, de2,
                dv_sh, de_sh, vidx_v, eidx_v, ones_v, semv, seme):
    cid = lax.axis_index("c")
    sid = lax.axis_index("s")
    pltpu.sync_copy(ones_hbm, ones_v)

    @pl.when(cid == 0)
    def _():
        _deg_stream(v1, e1, dv1, de1, zf, ones_v, NE1P,
                    dv_sh, de_sh, vidx_v, eidx_v, semv, seme, sid)

    @pl.when(cid == 1)
    def _():
        _deg_stream(v2, e2, dv2, de2, zf, ones_v, NE2P,
                    dv_sh, de_sh, vidx_v, eidx_v, semv, seme, sid)


def _ew_stream(ws_hbm, den_hbm, vidx_hbm, eidx_hbm, ew_hbm, den_rows,
               ws_v, den_v, vidx_v, eidx_v, ob, sid):
    pltpu.sync_copy(ws_hbm, ws_v)
    pltpu.sync_copy(den_hbm, den_v.at[pl.ds(0, den_rows)])

    @pl.loop(0, NCH)
    def _(ci):
        off = sid * PT + ci * K
        pltpu.sync_copy(vidx_hbm.at[pl.ds(off, K)], vidx_v)
        pltpu.sync_copy(eidx_hbm.at[pl.ds(off, K)], eidx_v)

        @pl.loop(0, K, step=16)
        def _(j):
            vi = vidx_v[pl.ds(j, 16)]
            ei = eidx_v[pl.ds(j, 16)]
            sp = plsc.load_gather(ws_v, [vi])
            dn = plsc.load_gather(den_v, [ei])
            ob[pl.ds(j, 16)] = sp / jnp.maximum(dn, 1e-12)

        pltpu.sync_copy(ob, ew_hbm.at[pl.ds(off, K)])


_EW_CP = pltpu.CompilerParams()
if "needs_layout_passes" in pltpu.CompilerParams.__dataclass_fields__:
    _EW_CP = dataclasses.replace(_EW_CP, needs_layout_passes=False)


@functools.partial(
    pl.kernel,
    out_type=[_sds((PP,)), _sds((PP,))],
    mesh=_MESH,
    compiler_params=_EW_CP,
    scratch_types=[
        pltpu.VMEM((N,), _f32),
        pltpu.VMEM((NE2P,), _f32),
        pltpu.VMEM((K,), jnp.int32),
        pltpu.VMEM((K,), jnp.int32),
        pltpu.VMEM((K,), _f32),
    ],
)
def _sc_ew(ws1, den1, v1, e1, ws2, den2, v2, e2, ew1, ew2,
           ws_v, den_v, vidx_v, eidx_v, ob):
    cid = lax.axis_index("c")
    sid = lax.axis_index("s")

    @pl.when(cid == 0)
    def _():
        _ew_stream(ws1, den1, v1, e1, ew1, NE1P,
                   ws_v, den_v, vidx_v, eidx_v, ob, sid)

    @pl.when(cid == 1)
    def _():
        _ew_stream(ws2, den2, v2, e2, ew2, NE2P,
                   ws_v, den_v, vidx_v, eidx_v, ob, sid)


def _isd_of(dv):
    d = dv[:, 0:1]
    return jnp.where(d > 0, 1.0 / jnp.sqrt(jnp.maximum(d, 1e-12)), 0.0)


def _xs_body(x_ref, w_ref, b_ref, dv_ref, o_ref):
    isd = _isd_of(dv_ref[...])
    o_ref[...] = (jnp.dot(x_ref[...], w_ref[...],
                          preferred_element_type=_f32) + b_ref[...]) * isd


def _tc_xs(X, W, b, dv):
    return pl.pallas_call(
        _xs_body,
        grid=(10,),
        in_specs=[
            pl.BlockSpec((1000, 128), lambda i: (i, 0)),
            pl.BlockSpec((128, 128), lambda i: (0, 0)),
            pl.BlockSpec((1, 128), lambda i: (0, 0)),
            pl.BlockSpec((1000, 128), lambda i: (i, 0)),
        ],
        out_specs=pl.BlockSpec((1000, 128), lambda i: (i, 0)),
        out_shape=_sds((N, 128)),
    )(X, W, b.reshape(1, 128), dv)


def _scale_de_body(ef_ref, de_ref, o_ref):
    d = de_ref[:, 0:1]
    o_ref[...] = ef_ref[...] * jnp.where(d > 0, 1.0 / jnp.maximum(d, 1e-12), 0.0)


def _tc_scale_de(ef, de, nep):
    return pl.pallas_call(
        _scale_de_body,
        out_shape=_sds((nep, 128)),
    )(ef, de)


def _h1xs2_body(va_ref, dv_ref, w_ref, b_ref, o_ref):
    isd = _isd_of(dv_ref[...])
    h1 = jnp.maximum(va_ref[...] * isd, 0.0)
    o_ref[...] = (jnp.dot(h1, w_ref[...],
                          preferred_element_type=_f32) + b_ref[...]) * isd


def _tc_h1xs2(va, dv, W23, b23):
    return pl.pallas_call(
        _h1xs2_body,
        grid=(10,),
        in_specs=[
            pl.BlockSpec((1000, 128), lambda i: (i, 0)),
            pl.BlockSpec((1000, 128), lambda i: (i, 0)),
            pl.BlockSpec((128, 128), lambda i: (0, 0)),
            pl.BlockSpec((1, 128), lambda i: (0, 0)),
        ],
        out_specs=pl.BlockSpec((1000, 128), lambda i: (i, 0)),
        out_shape=_sds((N, 128)),
    )(va, dv, W23, b23.reshape(1, 128))


def _scale_dv_body(va_ref, dv_ref, o_ref):
    o_ref[...] = va_ref[...] * _isd_of(dv_ref[...])


def _tc_scale_dv(va, dv):
    return pl.pallas_call(
        _scale_dv_body,
        grid=(10,),
        in_specs=[
            pl.BlockSpec((1000, 128), lambda i: (i, 0)),
            pl.BlockSpec((1000, 128), lambda i: (i, 0)),
        ],
        out_specs=pl.BlockSpec((1000, 128), lambda i: (i, 0)),
        out_shape=_sds((N, 128)),
    )(va, dv)


def _sb_body(m1_ref, m2_ref, wf_ref, bf_ref, o_ref):
    o_ref[...] = (
        jnp.dot(m1_ref[:, :64], wf_ref[:64, :], preferred_element_type=_f32)
        + jnp.dot(m2_ref[:, :64], wf_ref[64:, :], preferred_element_type=_f32)
        + bf_ref[...])


def _tc_sb(m1, m2, Wf, bf):
    return pl.pallas_call(
        _sb_body,
        grid=(10,),
        in_specs=[
            pl.BlockSpec((1000, 128), lambda i: (i, 0)),
            pl.BlockSpec((1000, 128), lambda i: (i, 0)),
            pl.BlockSpec((128, 64), lambda i: (0, 0)),
            pl.BlockSpec((1, 64), lambda i: (0, 0)),
        ],
        out_specs=pl.BlockSpec((1000, 64), lambda i: (i, 0)),
        out_shape=_sds((N, 64)),
    )(m1, m2, Wf, bf.reshape(1, 64))


def _attn_prep_body(sb_ref, wa_ref, ba_ref, va_ref, y_ref, ws_ref):
    t = jnp.tanh(jnp.dot(sb_ref[...], wa_ref[...],
                         preferred_element_type=_f32) + ba_ref[...])
    s = jnp.dot(t, va_ref[...], preferred_element_type=_f32)
    ws = jnp.exp(s - jnp.max(s))
    y_ref[...] = jnp.concatenate(
        [sb_ref[...] * ws, ws, jnp.zeros((N, 63), _f32)], axis=1)
    ws_ref[...] = ws


def _tc_attn_prep(sb, Wa, ba, va):
    wa_p = jnp.pad(Wa, ((0, 0), (0, 14)))
    ba_p = jnp.pad(ba, (0, 14)).reshape(1, 64)
    va_p = jnp.pad(va, (0, 14)).reshape(64, 1)
    return pl.pallas_call(
        _attn_prep_body,
        out_shape=[_sds((N, 128)), _sds((N, 1))],
    )(sb, wa_p, ba_p, va_p)


def _z_body(za_ref, z_ref, den_ref):
    den = za_ref[:, 64:65]
    z_ref[...] = za_ref[:, :64] / jnp.maximum(den, 1e-12)
    den_ref[...] = den


def _tc_z(za, nep):
    return pl.pallas_call(
        _z_body,
        out_shape=[_sds((nep, 64)), _sds((nep, 1))],
    )(za)


def _decode_body(sb_ref, z_ref, o_ref):
    h = lax.dot_general(sb_ref[...], z_ref[...],
                        (((1,), (1,)), ((), ())),
                        preferred_element_type=_f32)
    o_ref[...] = jax.nn.sigmoid(h)


def _tc_decode(sb, Z, ne):
    return pl.pallas_call(
        _decode_body,
        grid=(10,),
        in_specs=[
            pl.BlockSpec((1000, 64), lambda i: (i, 0)),
            pl.BlockSpec((ne, 64), lambda i: (0, 0)),
        ],
        out_specs=pl.BlockSpec((1000, ne), lambda i: (i, 0)),
        out_shape=_sds((N, ne)),
    )(sb, Z)


def kernel(X, params, hg1_v, hg1_e, hg2_v, hg2_e):
    pad = PP - P
    v1g = jnp.concatenate([hg1_v, jnp.zeros((pad,), jnp.int32)])
    v1s = jnp.concatenate([hg1_v, jnp.full((pad,), N, jnp.int32)])
    e1 = jnp.concatenate([hg1_e, jnp.full((pad,), NE1, jnp.int32)])
    v2g = jnp.concatenate([hg2_v, jnp.zeros((pad,), jnp.int32)])
    v2s = jnp.concatenate([hg2_v, jnp.full((pad,), N, jnp.int32)])
    e2 = jnp.concatenate([hg2_e, jnp.full((pad,), NE2, jnp.int32)])

    v1g2, v1s2, e1_2 = (a.reshape(-1, K) for a in (v1g, v1s, e1))
    v2g2, v2s2, e2_2 = (a.reshape(-1, K) for a in (v2g, v2s, e2))

    ones128 = jnp.ones((K, 128), _f32)
    zf128 = jnp.zeros((NVP // NS, 128), _f32)

    dv1, de1, dv2, de2 = _sc_degrees(v1s2, e1_2, v2s2, e2_2, ones128, zf128)

    p0, p1 = params["enc0"], params["enc1"]
    xs1_1 = _tc_xs(X, p0["W1"], p0["b1"], dv1)
    xs1_2 = _tc_xs(X, p1["W1"], p1["b1"], dv2)
    ef1, ef2 = _seg_edge(xs1_1, v1g2, e1_2, xs1_2, v2g2, e2_2, zf128)
    efs1 = _tc_scale_de(ef1, de1, NE1P)
    efs2 = _tc_scale_de(ef2, de2, NE2P)
    va1, va2 = _seg_vertex(efs1, e1_2, v1s2, efs2, e2_2, v2s2, zf128)

    W23_1 = jnp.concatenate([p0["W2"], p0["W3"]], axis=1)
    b23_1 = jnp.concatenate([p0["b2"], p0["b3"]])
    W23_2 = jnp.concatenate([p1["W2"], p1["W3"]], axis=1)
    b23_2 = jnp.concatenate([p1["b2"], p1["b3"]])
    xs2_1 = _tc_h1xs2(va1, dv1, W23_1, b23_1)
    xs2_2 = _tc_h1xs2(va2, dv2, W23_2, b23_2)
    eg1, eg2 = _seg_edge(xs2_1, v1g2, e1_2, xs2_2, v2g2, e2_2, zf128)
    egs1 = _tc_scale_de(eg1, de1, NE1P)
    egs2 = _tc_scale_de(eg2, de2, NE2P)
    vb1, vb2 = _seg_vertex(egs1, e1_2, v1s2, egs2, e2_2, v2s2, zf128)
    mulv1 = _tc_scale_dv(vb1, dv1)
    mulv2 = _tc_scale_dv(vb2, dv2)

    mu = jnp.concatenate([mulv1[:, :64], mulv2[:, :64]], axis=1)
    lv = jnp.concatenate([mulv1[:, 64:], mulv2[:, 64:]], axis=1)
    sb = _tc_sb(mulv1, mulv2, params["Wf"], params["bf"])

    a0, a1 = params["att0"], params["att1"]
    Y1, ws1 = _tc_attn_prep(sb, a0["Wa"], a0["ba"], a0["va"])
    Y2, ws2 = _tc_attn_prep(sb, a1["Wa"], a1["ba"], a1["va"])
    za1, za2 = _seg_attn(Y1, v1g2, e1_2, Y2, v2g2, e2_2, zf128)
    Z1f, den1 = _tc_z(za1, NE1P)
    Z2f, den2 = _tc_z(za2, NE2P)
    ew1p, ew2p = _sc_ew(ws1.reshape(N), den1.reshape(NE1P), v1g, e1,
                        ws2.reshape(N), den2.reshape(NE2P), v2g, e2)

    Z1 = Z1f[:NE1]
    Z2 = Z2f[:NE2]
    H1 = _tc_decode(sb, Z1, NE1)
    H2 = _tc_decode(sb, Z2, NE2)
    return (sb, Z1, Z2, H1, H2, mu, lv, ew1p[:P], ew2p[:P])


# speedup vs baseline: 1.0677x; 1.0677x over previous
import dataclasses
import functools

import jax
import jax.numpy as jnp
from jax import lax
from jax.experimental import pallas as pl
from jax.experimental.pallas import tpu as pltpu
from jax.experimental.pallas import tpu_sc as plsc

N = 10000
P = 320000
NE1, NE2 = 1024, 2048
NS = 16
K = 128
PT = 20480
PP = NS * PT
NCH = PT // K
NE1P, NE2P = 1280, 2304
NVP = 10240

_MESH = plsc.VectorSubcoreMesh(core_axis_name="c", subcore_axis_name="s")
_f32 = jnp.float32


def _sds(shape):
    return jax.ShapeDtypeStruct(shape, _f32)


GRP = 8


def _seg_stream(tbl_hbm, gidx_hbm, sidx_hbm, out_hbm, z_hbm, acc_rows,
                exp_rows, acc_sh, gidx_a, sidx_a, bufs, gsems, ssems,
                sid):
    ar = acc_rows // NS
    pltpu.sync_copy(z_hbm.at[pl.ds(0, ar)], acc_sh.at[pl.ds(sid * ar, ar)])
    plsc.subcore_barrier()

    @pl.loop(0, NCH // GRP)
    def _(g):
        base = sid * NCH + g * GRP
        pltpu.sync_copy(gidx_hbm.at[pl.ds(base, GRP)], gidx_a)
        pltpu.sync_copy(sidx_hbm.at[pl.ds(base, GRP)], sidx_a)
        nb_ = len(bufs)
        gobj = [None] * GRP
        sobj = [None] * GRP
        for b in range(nb_):
            gobj[b] = pltpu.async_copy(tbl_hbm.at[gidx_a.at[b]],
                                       bufs[b], gsems[b])
        for b in range(GRP):
            i = b % nb_
            gobj[b].wait()
            sobj[b] = pltpu.async_copy(bufs[i], acc_sh.at[sidx_a.at[b]],
                                       ssems[i], add=True)
            nxt = b + nb_
            if nxt < GRP:
                sobj[b].wait()
                gobj[nxt] = pltpu.async_copy(tbl_hbm.at[gidx_a.at[nxt]],
                                             bufs[i], gsems[i])
        for b in range(GRP - nb_, GRP):
            sobj[b].wait()

    plsc.subcore_barrier()
    er = exp_rows // NS
    pltpu.sync_copy(acc_sh.at[pl.ds(sid * er, er)],
                    out_hbm.at[pl.ds(sid * er, er)])


def _make_seg_kernel(C, accA_rows, accB_rows, expA_rows, expB_rows, NB=4):
    acc_max = max(accA_rows, accB_rows)

    @functools.partial(
        pl.kernel,
        out_type=[_sds((expA_rows, C)), _sds((expB_rows, C))],
        mesh=_MESH,
        scratch_types=[
            pltpu.VMEM_SHARED((acc_max, C), _f32),
            pltpu.VMEM((GRP, K), jnp.int32),
            pltpu.VMEM((GRP, K), jnp.int32),
        ] + [pltpu.VMEM((K, C), _f32)] * NB
          + [pltpu.SemaphoreType.DMA] * (2 * NB),
    )
    def seg(tblA, gA, sA, tblB, gB, sB, zf, outA, outB,
            acc_sh, gidx_a, sidx_a, *rest):
        cid = lax.axis_index("c")
        sid = lax.axis_index("s")
        bufs = tuple(rest[:NB])
        gsems = tuple(rest[NB:2 * NB])
        ssems = tuple(rest[2 * NB:])

        @pl.when(cid == 0)
        def _():
            _seg_stream(tblA, gA, sA, outA, zf, accA_rows, expA_rows,
                        acc_sh, gidx_a, sidx_a, bufs, gsems, ssems, sid)

        @pl.when(cid == 1)
        def _():
            _seg_stream(tblB, gB, sB, outB, zf, accB_rows, expB_rows,
                        acc_sh, gidx_a, sidx_a, bufs, gsems, ssems, sid)

    return seg


_seg_edge = _make_seg_kernel(128, NE1P, NE2P, NE1P, NE2P)
_seg_vertex = _make_seg_kernel(128, NVP, NVP, NVP, NVP, NB=2)
_seg_attn = _seg_edge


def _deg_stream(vidx_hbm, eidx_hbm, dv_hbm, de_hbm, z_hbm, ones_v,
                de_rows, dv_sh, de_sh, vidx_v, eidx_v, semv, seme, sid):
    dvr = NVP // NS
    pltpu.sync_copy(z_hbm.at[pl.ds(0, dvr)], dv_sh.at[pl.ds(sid * dvr, dvr)])
    der = de_rows // NS
    pltpu.sync_copy(z_hbm.at[pl.ds(0, der)], de_sh.at[pl.ds(sid * der, der)])
    plsc.subcore_barrier()

    @pl.loop(0, NCH // GRP)
    def _(g):
        base = sid * NCH + g * GRP
        pltpu.sync_copy(vidx_hbm.at[pl.ds(base, GRP)], vidx_v)
        pltpu.sync_copy(eidx_hbm.at[pl.ds(base, GRP)], eidx_v)
        objs = []
        for b in range(GRP):
            objs.append(pltpu.async_copy(
                ones_v, dv_sh.at[vidx_v.at[b]], semv, add=True))
            objs.append(pltpu.async_copy(
                ones_v, de_sh.at[eidx_v.at[b]], seme, add=True))
        for o in objs:
            o.wait()

    plsc.subcore_barrier()
    pltpu.sync_copy(dv_sh.at[pl.ds(sid * dvr, dvr)],
                    dv_hbm.at[pl.ds(sid * dvr, dvr)])
    pltpu.sync_copy(de_sh.at[pl.ds(sid * der, der)],
                    de_hbm.at[pl.ds(sid * der, der)])


@functools.partial(
    pl.kernel,
    out_type=[_sds((NVP, 128)), _sds((NE1P, 128)), _sds((NVP, 128)), _sds((NE2P, 128))],
    mesh=_MESH,
    scratch_types=[
        pltpu.VMEM_SHARED((NVP, 128), _f32),
        pltpu.VMEM_SHARED((NE2P, 128), _f32),
        pltpu.VMEM((GRP, K), jnp.int32),
        pltpu.VMEM((GRP, K), jnp.int32),
        pltpu.VMEM((K, 128), _f32),
        pltpu.SemaphoreType.DMA,
        pltpu.SemaphoreType.DMA,
    ],
)
def _sc_degrees(v1, e1, v2, e2, ones_hbm, zf, dv1, de1, dv2, de2,
                dv_sh, de_sh, vidx_v, eidx_v, ones_v, semv, seme):
    cid = lax.axis_index("c")
    sid = lax.axis_index("s")
    pltpu.sync_copy(ones_hbm, ones_v)

    @pl.when(cid == 0)
    def _():
        _deg_stream(v1, e1, dv1, de1, zf, ones_v, NE1P,
                    dv_sh, de_sh, vidx_v, eidx_v, semv, seme, sid)

    @pl.when(cid == 1)
    def _():
        _deg_stream(v2, e2, dv2, de2, zf, ones_v, NE2P,
                    dv_sh, de_sh, vidx_v, eidx_v, semv, seme, sid)


def _ew_stream(ws_hbm, den_hbm, vidx_hbm, eidx_hbm, ew_hbm, den_rows,
               ws_v, den_v, vidx_a, eidx_a, ob, sid):
    pltpu.sync_copy(ws_hbm, ws_v)
    pltpu.sync_copy(den_hbm, den_v.at[pl.ds(0, den_rows)])

    @pl.loop(0, NCH // GRP)
    def _(g):
        base = sid * NCH + g * GRP
        pltpu.sync_copy(vidx_hbm.at[pl.ds(base, GRP)], vidx_a)
        pltpu.sync_copy(eidx_hbm.at[pl.ds(base, GRP)], eidx_a)
        for b in range(GRP):
            @pl.loop(0, K, step=16)
            def _(j):
                vi = vidx_a[b, pl.ds(j, 16)]
                ei = eidx_a[b, pl.ds(j, 16)]
                sp = plsc.load_gather(ws_v, [vi])
                dn = plsc.load_gather(den_v, [ei])
                ob[b, pl.ds(j, 16)] = sp / jnp.maximum(dn, 1e-12)
        pltpu.sync_copy(ob, ew_hbm.at[pl.ds(base, GRP)])


_EW_CP = pltpu.CompilerParams()
if "needs_layout_passes" in pltpu.CompilerParams.__dataclass_fields__:
    _EW_CP = dataclasses.replace(_EW_CP, needs_layout_passes=False)


@functools.partial(
    pl.kernel,
    out_type=[_sds((PP // K, K)), _sds((PP // K, K))],
    mesh=_MESH,
    compiler_params=_EW_CP,
    scratch_types=[
        pltpu.VMEM((N,), _f32),
        pltpu.VMEM((NE2P,), _f32),
        pltpu.VMEM((GRP, K), jnp.int32),
        pltpu.VMEM((GRP, K), jnp.int32),
        pltpu.VMEM((GRP, K), _f32),
    ],
)
def _sc_ew(ws1, den1, v1, e1, ws2, den2, v2, e2, ew1, ew2,
           ws_v, den_v, vidx_v, eidx_v, ob):
    cid = lax.axis_index("c")
    sid = lax.axis_index("s")

    @pl.when(cid == 0)
    def _():
        _ew_stream(ws1, den1, v1, e1, ew1, NE1P,
                   ws_v, den_v, vidx_v, eidx_v, ob, sid)

    @pl.when(cid == 1)
    def _():
        _ew_stream(ws2, den2, v2, e2, ew2, NE2P,
                   ws_v, den_v, vidx_v, eidx_v, ob, sid)


def _isd_of(dv):
    d = dv[:, 0:1]
    return jnp.where(d > 0, 1.0 / jnp.sqrt(jnp.maximum(d, 1e-12)), 0.0)


def _xs_body(x_ref, w_ref, b_ref, dv_ref, o_ref):
    isd = _isd_of(dv_ref[...])
    o_ref[...] = (jnp.dot(x_ref[...], w_ref[...],
                          preferred_element_type=_f32) + b_ref[...]) * isd


def _tc_xs(X, W, b, dv):
    return pl.pallas_call(
        _xs_body,
        grid=(10,),
        in_specs=[
            pl.BlockSpec((1000, 128), lambda i: (i, 0)),
            pl.BlockSpec((128, 128), lambda i: (0, 0)),
            pl.BlockSpec((1, 128), lambda i: (0, 0)),
            pl.BlockSpec((1000, 128), lambda i: (i, 0)),
        ],
        out_specs=pl.BlockSpec((1000, 128), lambda i: (i, 0)),
        out_shape=_sds((N, 128)),
    )(X, W, b.reshape(1, 128), dv)


def _scale_de_body(ef_ref, de_ref, o_ref):
    d = de_ref[:, 0:1]
    o_ref[...] = ef_ref[...] * jnp.where(d > 0, 1.0 / jnp.maximum(d, 1e-12), 0.0)


def _tc_scale_de(ef, de, nep):
    return pl.pallas_call(
        _scale_de_body,
        out_shape=_sds((nep, 128)),
    )(ef, de)


def _h1xs2_body(va_ref, dv_ref, w_ref, b_ref, o_ref):
    isd = _isd_of(dv_ref[...])
    h1 = jnp.maximum(va_ref[...] * isd, 0.0)
    o_ref[...] = (jnp.dot(h1, w_ref[...],
                          preferred_element_type=_f32) + b_ref[...]) * isd


def _tc_h1xs2(va, dv, W23, b23):
    return pl.pallas_call(
        _h1xs2_body,
        grid=(10,),
        in_specs=[
            pl.BlockSpec((1000, 128), lambda i: (i, 0)),
            pl.BlockSpec((1000, 128), lambda i: (i, 0)),
            pl.BlockSpec((128, 128), lambda i: (0, 0)),
            pl.BlockSpec((1, 128), lambda i: (0, 0)),
        ],
        out_specs=pl.BlockSpec((1000, 128), lambda i: (i, 0)),
        out_shape=_sds((N, 128)),
    )(va, dv, W23, b23.reshape(1, 128))


def _scale_dv_body(va_ref, dv_ref, o_ref):
    o_ref[...] = va_ref[...] * _isd_of(dv_ref[...])


def _tc_scale_dv(va, dv):
    return pl.pallas_call(
        _scale_dv_body,
        grid=(10,),
        in_specs=[
            pl.BlockSpec((1000, 128), lambda i: (i, 0)),
            pl.BlockSpec((1000, 128), lambda i: (i, 0)),
        ],
        out_specs=pl.BlockSpec((1000, 128), lambda i: (i, 0)),
        out_shape=_sds((N, 128)),
    )(va, dv)


def _sb_body(m1_ref, m2_ref, wf_ref, bf_ref, o_ref):
    o_ref[...] = (
        jnp.dot(m1_ref[:, :64], wf_ref[:64, :], preferred_element_type=_f32)
        + jnp.dot(m2_ref[:, :64], wf_ref[64:, :], preferred_element_type=_f32)
        + bf_ref[...])


def _tc_sb(m1, m2, Wf, bf):
    return pl.pallas_call(
        _sb_body,
        grid=(10,),
        in_specs=[
            pl.BlockSpec((1000, 128), lambda i: (i, 0)),
            pl.BlockSpec((1000, 128), lambda i: (i, 0)),
            pl.BlockSpec((128, 64), lambda i: (0, 0)),
            pl.BlockSpec((1, 64), lambda i: (0, 0)),
        ],
        out_specs=pl.BlockSpec((1000, 64), lambda i: (i, 0)),
        out_shape=_sds((N, 64)),
    )(m1, m2, Wf, bf.reshape(1, 64))


def _attn_prep_body(sb_ref, wa_ref, ba_ref, va_ref, y_ref, ws_ref):
    t = jnp.tanh(jnp.dot(sb_ref[...], wa_ref[...],
                         preferred_element_type=_f32) + ba_ref[...])
    s = jnp.dot(t, va_ref[...], preferred_element_type=_f32)
    ws = jnp.exp(s - jnp.max(s))
    y_ref[...] = jnp.concatenate(
        [sb_ref[...] * ws, ws, jnp.zeros((N, 63), _f32)], axis=1)
    ws_ref[...] = ws


def _tc_attn_prep(sb, Wa, ba, va):
    wa_p = jnp.pad(Wa, ((0, 0), (0, 14)))
    ba_p = jnp.pad(ba, (0, 14)).reshape(1, 64)
    va_p = jnp.pad(va, (0, 14)).reshape(64, 1)
    return pl.pallas_call(
        _attn_prep_body,
        out_shape=[_sds((N, 128)), _sds((N, 1))],
    )(sb, wa_p, ba_p, va_p)


def _z_body(za_ref, z_ref, den_ref):
    den = za_ref[:, 64:65]
    z_ref[...] = za_ref[:, :64] / jnp.maximum(den, 1e-12)
    den_ref[...] = den


def _tc_z(za, nep):
    return pl.pallas_call(
        _z_body,
        out_shape=[_sds((nep, 64)), _sds((nep, 1))],
    )(za)


def _decode_body(sb_ref, z_ref, o_ref):
    h = lax.dot_general(sb_ref[...], z_ref[...],
                        (((1,), (1,)), ((), ())),
                        preferred_element_type=_f32)
    o_ref[...] = jax.nn.sigmoid(h)


def _tc_decode(sb, Z, ne):
    return pl.pallas_call(
        _decode_body,
        grid=(10,),
        in_specs=[
            pl.BlockSpec((1000, 64), lambda i: (i, 0)),
            pl.BlockSpec((ne, 64), lambda i: (0, 0)),
        ],
        out_specs=pl.BlockSpec((1000, ne), lambda i: (i, 0)),
        out_shape=_sds((N, ne)),
    )(sb, Z)


def kernel(X, params, hg1_v, hg1_e, hg2_v, hg2_e):
    pad = PP - P
    v1g = jnp.concatenate([hg1_v, jnp.zeros((pad,), jnp.int32)])
    v1s = jnp.concatenate([hg1_v, jnp.full((pad,), N, jnp.int32)])
    e1 = jnp.concatenate([hg1_e, jnp.full((pad,), NE1, jnp.int32)])
    v2g = jnp.concatenate([hg2_v, jnp.zeros((pad,), jnp.int32)])
    v2s = jnp.concatenate([hg2_v, jnp.full((pad,), N, jnp.int32)])
    e2 = jnp.concatenate([hg2_e, jnp.full((pad,), NE2, jnp.int32)])

    v1g2, v1s2, e1_2 = (a.reshape(-1, K) for a in (v1g, v1s, e1))
    v2g2, v2s2, e2_2 = (a.reshape(-1, K) for a in (v2g, v2s, e2))

    ones128 = jnp.ones((K, 128), _f32)
    zf128 = jnp.zeros((NVP // NS, 128), _f32)

    dv1, de1, dv2, de2 = _sc_degrees(v1s2, e1_2, v2s2, e2_2, ones128, zf128)

    p0, p1 = params["enc0"], params["enc1"]
    xs1_1 = _tc_xs(X, p0["W1"], p0["b1"], dv1)
    xs1_2 = _tc_xs(X, p1["W1"], p1["b1"], dv2)
    ef1, ef2 = _seg_edge(xs1_1, v1g2, e1_2, xs1_2, v2g2, e2_2, zf128)
    efs1 = _tc_scale_de(ef1, de1, NE1P)
    efs2 = _tc_scale_de(ef2, de2, NE2P)
    va1, va2 = _seg_vertex(efs1, e1_2, v1s2, efs2, e2_2, v2s2, zf128)

    W23_1 = jnp.concatenate([p0["W2"], p0["W3"]], axis=1)
    b23_1 = jnp.concatenate([p0["b2"], p0["b3"]])
    W23_2 = jnp.concatenate([p1["W2"], p1["W3"]], axis=1)
    b23_2 = jnp.concatenate([p1["b2"], p1["b3"]])
    xs2_1 = _tc_h1xs2(va1, dv1, W23_1, b23_1)
    xs2_2 = _tc_h1xs2(va2, dv2, W23_2, b23_2)
    eg1, eg2 = _seg_edge(xs2_1, v1g2, e1_2, xs2_2, v2g2, e2_2, zf128)
    egs1 = _tc_scale_de(eg1, de1, NE1P)
    egs2 = _tc_scale_de(eg2, de2, NE2P)
    vb1, vb2 = _seg_vertex(egs1, e1_2, v1s2, egs2, e2_2, v2s2, zf128)
    mulv1 = _tc_scale_dv(vb1, dv1)
    mulv2 = _tc_scale_dv(vb2, dv2)

    mu = jnp.concatenate([mulv1[:, :64], mulv2[:, :64]], axis=1)
    lv = jnp.concatenate([mulv1[:, 64:], mulv2[:, 64:]], axis=1)
    sb = _tc_sb(mulv1, mulv2, params["Wf"], params["bf"])

    a0, a1 = params["att0"], params["att1"]
    Y1, ws1 = _tc_attn_prep(sb, a0["Wa"], a0["ba"], a0["va"])
    Y2, ws2 = _tc_attn_prep(sb, a1["Wa"], a1["ba"], a1["va"])
    za1, za2 = _seg_attn(Y1, v1g2, e1_2, Y2, v2g2, e2_2, zf128)
    Z1f, den1 = _tc_z(za1, NE1P)
    Z2f, den2 = _tc_z(za2, NE2P)
    ew1p, ew2p = _sc_ew(ws1.reshape(N), den1.reshape(NE1P), v1g2, e1_2,
                        ws2.reshape(N), den2.reshape(NE2P), v2g2, e2_2)
    ew1p = ew1p.reshape(PP)
    ew2p = ew2p.reshape(PP)

    Z1 = Z1f[:NE1]
    Z2 = Z2f[:NE2]
    H1 = _tc_decode(sb, Z1, NE1)
    H2 = _tc_decode(sb, Z2, NE2)
    return (sb, Z1, Z2, H1, H2, mu, lv, ew1p[:P], ew2p[:P])
